# Initial kernel scaffold; baseline (speedup 1.0000x reference)
import jax
import jax.numpy as jnp
from jax.experimental import pallas as pl


def kernel(x, edge_index, W1, b1, W2, b2):
    raise NotImplementedError("write your pallas kernel here")


# speedup vs baseline: 7.6006x; 7.6006x over previous
import functools

import jax
import jax.numpy as jnp
from jax import lax
from jax.experimental import pallas as pl
from jax.experimental.pallas import tpu as pltpu
from jax.experimental.pallas import tpu_sc as plsc

N = 10000
D = 128
DH = D // 2
NP = 10240
NC = 2
NS = 16
NW = NC * NS
EB = 128
NBLK = 80
EPAD = NW * NBLK * EB
RPT = NP // NS

_mesh = plsc.VectorSubcoreMesh(core_axis_name="c", subcore_axis_name="s")


@functools.partial(
    pl.kernel,
    out_type=jax.ShapeDtypeStruct((NC * NP, 16), jnp.float32),
    mesh=_mesh,
    scratch_types=[
        pltpu.VMEM((EB, 16), jnp.float32),
        pltpu.VMEM((EB, 16), jnp.float32),
        pltpu.VMEM((NBLK, EB), jnp.int32),
        pltpu.VMEM_SHARED((NP, 16), jnp.float32),
    ],
    compiler_params=pltpu.CompilerParams(use_tc_tiling_on_sc=False),
)
def _deg_kernel(dst_hbm, out_hbm, ones_v, zero_v, didx, acc):
    c = lax.axis_index("c")
    s = lax.axis_index("s")
    wid = c * NS + s

    @pl.loop(0, EB)
    def _fill(r):
        ones_v[r, :] = jnp.ones((16,), jnp.float32)
        zero_v[r, :] = jnp.zeros((16,), jnp.float32)

    for k in range(RPT // EB):
        pltpu.sync_copy(zero_v, acc.at[pl.ds(s * RPT + k * EB, EB)])
    plsc.subcore_barrier()

    pltpu.sync_copy(dst_hbm.at[pl.ds(wid * NBLK, NBLK)], didx)

    @pl.loop(0, NBLK)
    def _hist(i):
        pltpu.sync_copy(ones_v, acc.at[didx.at[i]], add=True)

    plsc.subcore_barrier()
    pltpu.sync_copy(acc.at[pl.ds(s * RPT, RPT)],
                    out_hbm.at[pl.ds(c * NP + s * RPT, RPT)])


@functools.partial(
    pl.kernel,
    out_type=[jax.ShapeDtypeStruct((NC * NP, DH), jnp.float32),
              jax.ShapeDtypeStruct((NC * NP, DH), jnp.float32)],
    mesh=_mesh,
    scratch_types=[
        pltpu.VMEM((NBLK + 1, EB), jnp.int32),
        pltpu.VMEM((NBLK, EB), jnp.int32),
        pltpu.VMEM((EB, DH), jnp.float32),
        pltpu.VMEM((EB, DH), jnp.float32),
        pltpu.VMEM((EB, DH), jnp.float32),
        pltpu.VMEM_SHARED((NP, DH), jnp.float32),
        pltpu.SemaphoreType.DMA,
        pltpu.SemaphoreType.DMA,
    ],
    compiler_params=pltpu.CompilerParams(use_tc_tiling_on_sc=False),
)
def _agg_kernel(h0_hbm, h1_hbm, src_hbm, dst_hbm, out0_hbm, out1_hbm,
                sidx, didx, rows0, rows1, zbuf, acc, gsem0, gsem1):
    c = lax.axis_index("c")
    s = lax.axis_index("s")
    wid = c * NS + s

    @pl.loop(0, EB)
    def _zero(r):
        for j in range(DH // 16):
            zbuf[r, pl.ds(j * 16, 16)] = jnp.zeros((16,), jnp.float32)

    for j in range(EB // 16):
        sidx[NBLK, pl.ds(j * 16, 16)] = jnp.zeros((16,), jnp.int32)

    pltpu.sync_copy(src_hbm.at[pl.ds(wid * NBLK, NBLK)],
                    sidx.at[pl.ds(0, NBLK)])
    pltpu.sync_copy(dst_hbm.at[pl.ds(wid * NBLK, NBLK)], didx)

    for h_hbm, out_hbm in ((h0_hbm, out0_hbm), (h1_hbm, out1_hbm)):
        for k in range(RPT // EB):
            pltpu.sync_copy(zbuf, acc.at[pl.ds(s * RPT + k * EB, EB)])
        plsc.subcore_barrier()

        pltpu.async_copy(h_hbm.at[sidx.at[0]], rows0, gsem0)

        @pl.loop(0, NBLK, step=2)
        def _body(i):
            pltpu.async_copy(h_hbm.at[sidx.at[i + 1]], rows1, gsem1)
            pltpu.make_async_copy(h_hbm.at[sidx.at[i]], rows0, gsem0).wait()
            pltpu.sync_copy(rows0, acc.at[didx.at[i]], add=True)
            pltpu.async_copy(h_hbm.at[sidx.at[i + 2]], rows0, gsem0)
            pltpu.make_async_copy(h_hbm.at[sidx.at[i + 1]], rows1,
                                  gsem1).wait()
            pltpu.sync_copy(rows1, acc.at[didx.at[i + 1]], add=True)

        pltpu.make_async_copy(h_hbm.at[sidx.at[0]], rows0, gsem0).wait()

        plsc.subcore_barrier()
        pltpu.sync_copy(acc.at[pl.ds(s * RPT, RPT)],
                        out_hbm.at[pl.ds(c * NP + s * RPT, RPT)])
        plsc.subcore_barrier()


BR = 256


def _mm_scale_body(x_ref, w_ref, deg_ref, o0_ref, o1_ref):
    dis = lax.rsqrt(deg_ref[...])
    h = jnp.dot(x_ref[...], w_ref[...], preferred_element_type=jnp.float32)
    h = h * dis
    o0_ref[...] = h[:, :DH]
    o1_ref[...] = h[:, DH:]


def _mm_scale(xp, W, degcol):
    return pl.pallas_call(
        _mm_scale_body,
        grid=(NP // BR,),
        in_specs=[
            pl.BlockSpec((BR, D), lambda i: (i, 0)),
            pl.BlockSpec((D, D), lambda i: (0, 0)),
            pl.BlockSpec((BR, 1), lambda i: (i, 0)),
        ],
        out_specs=[pl.BlockSpec((BR, DH), lambda i: (i, 0)),
                   pl.BlockSpec((BR, DH), lambda i: (i, 0))],
        out_shape=[jax.ShapeDtypeStruct((NP, DH), jnp.float32),
                   jax.ShapeDtypeStruct((NP, DH), jnp.float32)],
    )(xp, W, degcol)


def _mid_body(p0_ref, p1_ref, h0_ref, h1_ref, deg_ref, b_ref, w_ref,
              o0_ref, o1_ref):
    dis = lax.rsqrt(deg_ref[...])
    agg0 = p0_ref[0] + p0_ref[1] + h0_ref[...]
    agg1 = p1_ref[0] + p1_ref[1] + h1_ref[...]
    agg = jnp.concatenate([agg0, agg1], axis=1)
    z = jnp.maximum(agg * dis + b_ref[...], 0.0)
    h2 = jnp.dot(z, w_ref[...], preferred_element_type=jnp.float32)
    h2 = h2 * dis
    o0_ref[...] = h2[:, :DH]
    o1_ref[...] = h2[:, DH:]


def _mid(p0, p1, h0, h1, degcol, b1, W2):
    return pl.pallas_call(
        _mid_body,
        grid=(NP // BR,),
        in_specs=[
            pl.BlockSpec((2, BR, DH), lambda i: (0, i, 0)),
            pl.BlockSpec((2, BR, DH), lambda i: (0, i, 0)),
            pl.BlockSpec((BR, DH), lambda i: (i, 0)),
            pl.BlockSpec((BR, DH), lambda i: (i, 0)),
            pl.BlockSpec((BR, 1), lambda i: (i, 0)),
            pl.BlockSpec((1, D), lambda i: (0, 0)),
            pl.BlockSpec((D, D), lambda i: (0, 0)),
        ],
        out_specs=[pl.BlockSpec((BR, DH), lambda i: (i, 0)),
                   pl.BlockSpec((BR, DH), lambda i: (i, 0))],
        out_shape=[jax.ShapeDtypeStruct((NP, DH), jnp.float32),
                   jax.ShapeDtypeStruct((NP, DH), jnp.float32)],
    )(p0, p1, h0, h1, degcol, b1, W2)


def _final_body(p0_ref, p1_ref, h0_ref, h1_ref, deg_ref, b_ref, o_ref):
    dis = lax.rsqrt(deg_ref[...])
    agg0 = p0_ref[0] + p0_ref[1] + h0_ref[...]
    agg1 = p1_ref[0] + p1_ref[1] + h1_ref[...]
    agg = jnp.concatenate([agg0, agg1], axis=1)
    o_ref[...] = agg * dis + b_ref[...]


def _final(p0, p1, h0, h1, degcol, b2):
    return pl.pallas_call(
        _final_body,
        grid=(NP // BR,),
        in_specs=[
            pl.BlockSpec((2, BR, DH), lambda i: (0, i, 0)),
            pl.BlockSpec((2, BR, DH), lambda i: (0, i, 0)),
            pl.BlockSpec((BR, DH), lambda i: (i, 0)),
            pl.BlockSpec((BR, DH), lambda i: (i, 0)),
            pl.BlockSpec((BR, 1), lambda i: (i, 0)),
            pl.BlockSpec((1, D), lambda i: (0, 0)),
        ],
        out_specs=pl.BlockSpec((BR, D), lambda i: (i, 0)),
        out_shape=jax.ShapeDtypeStruct((NP, D), jnp.float32),
    )(p0, p1, h0, h1, degcol, b2)


def kernel(x, edge_index, W1, b1, W2, b2):
    E = edge_index.shape[1]
    pad = EPAD - E
    src = jnp.concatenate(
        [edge_index[0], jnp.zeros((pad,), edge_index.dtype)]).reshape(
            NW * NBLK, EB)
    dst = jnp.concatenate(
        [edge_index[1], jnp.full((pad,), N, edge_index.dtype)]).reshape(
            NW * NBLK, EB)
    xp = jnp.pad(x, ((0, NP - N), (0, 0)))

    degp = _deg_kernel(dst).reshape(NC, NP, 16)
    degcol = degp[0, :, 0:1] + degp[1, :, 0:1] + 1.0

    h1a, h1b = _mm_scale(xp, W1, degcol)
    q0, q1 = _agg_kernel(h1a, h1b, src, dst)
    h2a, h2b = _mid(q0.reshape(NC, NP, DH), q1.reshape(NC, NP, DH),
                    h1a, h1b, degcol, b1[None, :], W2)
    q0, q1 = _agg_kernel(h2a, h2b, src, dst)
    out = _final(q0.reshape(NC, NP, DH), q1.reshape(NC, NP, DH),
                 h2a, h2b, degcol, b2[None, :])
    return out[:N]
